# baseline (speedup 1.0000x reference)
import functools

import jax
import jax.numpy as jnp
from jax import lax
from jax.experimental import pallas as pl

_B = 16384
_DX = 2213
_NRATE = 6
_NYEAR = 81
_EMB = 32
_BB = 1024
_NCHUNK = 9
_CW = 256


def _body(*refs):
    x_refs = refs[:_NCHUNK]
    w_refs = refs[_NCHUNK:2 * _NCHUNK]
    rate_ref, year_ref, out_ref = refs[2 * _NCHUNK:]

    pre = jnp.zeros((_BB, 2 * _EMB), jnp.float32)
    for xr, wr in zip(x_refs, w_refs):
        xf = xr[...].astype(jnp.bfloat16)
        pre = pre + jnp.dot(xf, wr[...], preferred_element_type=jnp.float32)
    proj = jax.nn.sigmoid(pre)

    x01 = x_refs[0][...]
    oh_rate = (x01[:, 0:1] == lax.broadcasted_iota(jnp.int32, (_BB, _NRATE), 1)
               ).astype(jnp.float32)
    oh_year = (x01[:, 1:2] == lax.broadcasted_iota(jnp.int32, (_BB, _NYEAR), 1)
               ).astype(jnp.float32)
    rate_emb = jnp.dot(oh_rate, rate_ref[...], preferred_element_type=jnp.float32)
    year_emb = jnp.dot(oh_year, year_ref[...], preferred_element_type=jnp.float32)

    out_ref[...] = jnp.concatenate([rate_emb, year_emb, proj], axis=1)


def _build(interpret=False):
    x_specs = [
        pl.BlockSpec((_BB, _CW), functools.partial(lambda j, i: (i, j), j))
        for j in range(_NCHUNK)
    ]
    w_specs = [pl.BlockSpec((_CW, 2 * _EMB), lambda i: (0, 0)) for _ in range(_NCHUNK)]
    return pl.pallas_call(
        _body,
        grid=(_B // _BB,),
        in_specs=x_specs + w_specs + [
            pl.BlockSpec((_NRATE, _EMB), lambda i: (0, 0)),
            pl.BlockSpec((_NYEAR, _EMB), lambda i: (0, 0)),
        ],
        out_specs=pl.BlockSpec((_BB, 4 * _EMB), lambda i: (i, 0)),
        out_shape=jax.ShapeDtypeStruct((_B, 4 * _EMB), jnp.float32),
        interpret=interpret,
    )


def kernel(x, rate_table, year_table, W_genre, W_director):
    wbig = jnp.zeros((_NCHUNK * _CW, 2 * _EMB), jnp.float32)
    wbig = wbig.at[2:27, 0:_EMB].set(W_genre.T)
    wbig = wbig.at[27:_DX, _EMB:].set(W_director.T)
    wbig = wbig.astype(jnp.bfloat16)
    wchunks = [wbig[j * _CW:(j + 1) * _CW] for j in range(_NCHUNK)]
    return _build()(*([x] * _NCHUNK), *wchunks, rate_table, year_table)


# speedup vs baseline: 1.0005x; 1.0005x over previous
import functools

import jax
import jax.numpy as jnp
from jax import lax
from jax.experimental import pallas as pl

_B = 16384
_DX = 2213
_NRATE = 6
_NYEAR = 81
_EMB = 32
_BB = 1024
_NCHUNK = 3
_CW = 768


def _body(*refs):
    x_refs = refs[:_NCHUNK]
    w_refs = refs[_NCHUNK:2 * _NCHUNK]
    rate_ref, year_ref, out_ref = refs[2 * _NCHUNK:]

    pre = jnp.zeros((_BB, 2 * _EMB), jnp.float32)
    for xr, wr in zip(x_refs, w_refs):
        xf = xr[...].astype(jnp.bfloat16)
        pre = pre + jnp.dot(xf, wr[...], preferred_element_type=jnp.float32)
    proj = jax.nn.sigmoid(pre)

    x01 = x_refs[0][...]
    oh_rate = (x01[:, 0:1] == lax.broadcasted_iota(jnp.int32, (_BB, _NRATE), 1)
               ).astype(jnp.float32)
    oh_year = (x01[:, 1:2] == lax.broadcasted_iota(jnp.int32, (_BB, _NYEAR), 1)
               ).astype(jnp.float32)
    rate_emb = jnp.dot(oh_rate, rate_ref[...], preferred_element_type=jnp.float32)
    year_emb = jnp.dot(oh_year, year_ref[...], preferred_element_type=jnp.float32)

    out_ref[...] = jnp.concatenate([rate_emb, year_emb, proj], axis=1)


def _build(interpret=False):
    x_specs = [
        pl.BlockSpec((_BB, _CW), functools.partial(lambda j, i: (i, j), j))
        for j in range(_NCHUNK)
    ]
    w_specs = [pl.BlockSpec((_CW, 2 * _EMB), lambda i: (0, 0)) for _ in range(_NCHUNK)]
    return pl.pallas_call(
        _body,
        grid=(_B // _BB,),
        in_specs=x_specs + w_specs + [
            pl.BlockSpec((_NRATE, _EMB), lambda i: (0, 0)),
            pl.BlockSpec((_NYEAR, _EMB), lambda i: (0, 0)),
        ],
        out_specs=pl.BlockSpec((_BB, 4 * _EMB), lambda i: (i, 0)),
        out_shape=jax.ShapeDtypeStruct((_B, 4 * _EMB), jnp.float32),
        interpret=interpret,
    )


def kernel(x, rate_table, year_table, W_genre, W_director):
    wbig = jnp.zeros((_NCHUNK * _CW, 2 * _EMB), jnp.float32)
    wbig = wbig.at[2:27, 0:_EMB].set(W_genre.T)
    wbig = wbig.at[27:_DX, _EMB:].set(W_director.T)
    wbig = wbig.astype(jnp.bfloat16)
    wchunks = [wbig[j * _CW:(j + 1) * _CW] for j in range(_NCHUNK)]
    return _build()(*([x] * _NCHUNK), *wchunks, rate_table, year_table)


# speedup vs baseline: 1.0009x; 1.0004x over previous
import functools

import jax
import jax.numpy as jnp
from jax import lax
from jax.experimental import pallas as pl

_B = 16384
_DX = 2213
_NRATE = 6
_NYEAR = 81
_EMB = 32
_BB = 1024
_NCHUNK = 6
_CW = 384


def _body(*refs):
    x_refs = refs[:_NCHUNK]
    w_refs = refs[_NCHUNK:2 * _NCHUNK]
    rate_ref, year_ref, out_ref = refs[2 * _NCHUNK:]

    pre = jnp.zeros((_BB, 2 * _EMB), jnp.float32)
    for xr, wr in zip(x_refs, w_refs):
        xf = xr[...].astype(jnp.bfloat16)
        pre = pre + jnp.dot(xf, wr[...], preferred_element_type=jnp.float32)
    proj = jax.nn.sigmoid(pre)

    x01 = x_refs[0][...]
    oh_rate = (x01[:, 0:1] == lax.broadcasted_iota(jnp.int32, (_BB, _NRATE), 1)
               ).astype(jnp.float32)
    oh_year = (x01[:, 1:2] == lax.broadcasted_iota(jnp.int32, (_BB, _NYEAR), 1)
               ).astype(jnp.float32)
    rate_emb = jnp.dot(oh_rate, rate_ref[...], preferred_element_type=jnp.float32)
    year_emb = jnp.dot(oh_year, year_ref[...], preferred_element_type=jnp.float32)

    out_ref[...] = jnp.concatenate([rate_emb, year_emb, proj], axis=1)


def _build(interpret=False):
    x_specs = [
        pl.BlockSpec((_BB, _CW), functools.partial(lambda j, i: (i, j), j))
        for j in range(_NCHUNK)
    ]
    w_specs = [pl.BlockSpec((_CW, 2 * _EMB), lambda i: (0, 0)) for _ in range(_NCHUNK)]
    return pl.pallas_call(
        _body,
        grid=(_B // _BB,),
        in_specs=x_specs + w_specs + [
            pl.BlockSpec((_NRATE, _EMB), lambda i: (0, 0)),
            pl.BlockSpec((_NYEAR, _EMB), lambda i: (0, 0)),
        ],
        out_specs=pl.BlockSpec((_BB, 4 * _EMB), lambda i: (i, 0)),
        out_shape=jax.ShapeDtypeStruct((_B, 4 * _EMB), jnp.float32),
        interpret=interpret,
    )


def kernel(x, rate_table, year_table, W_genre, W_director):
    wbig = jnp.zeros((_NCHUNK * _CW, 2 * _EMB), jnp.float32)
    wbig = wbig.at[2:27, 0:_EMB].set(W_genre.T)
    wbig = wbig.at[27:_DX, _EMB:].set(W_director.T)
    wbig = wbig.astype(jnp.bfloat16)
    wchunks = [wbig[j * _CW:(j + 1) * _CW] for j in range(_NCHUNK)]
    return _build()(*([x] * _NCHUNK), *wchunks, rate_table, year_table)
